# baseline (speedup 1.0000x reference)
import functools

import jax
import jax.numpy as jnp
from jax import lax
from jax.experimental import pallas as pl
from jax.experimental.pallas import tpu as pltpu
from jax.experimental.pallas import tpu_sc as plsc

B = 2
NSPAN = 512
D = 1024
ROWS = 64
NSP = B * NSPAN
NW = 32
SPW = NSP // NW
HALF = SPW // 2


def _prep_body(seq_ref, st_ref, en_ref, stb_ref, enb_ref,
               p_ref, idxa_ref, idxb_ref, ag_ref):
    k = lax.broadcasted_iota(jnp.int32, (B * ROWS, B * ROWS), 0)
    p = lax.broadcasted_iota(jnp.int32, (B * ROWS, B * ROWS), 1)
    tri = ((p < k) & ((p // ROWS) == (k // ROWS))).astype(jnp.float32)
    p_ref[...] = lax.dot_general(
        tri, seq_ref[...], (((1,), (0,)), ((), ())),
        preferred_element_type=jnp.float32)

    e = en_ref[...] - 1
    w = e - st_ref[...]
    wmax = jnp.max(w) + 1
    lcnt = jnp.where(w >= 0, w + 1, wmax)
    lo = e - lcnt + 1
    boff = jnp.where(
        lax.broadcasted_iota(jnp.int32, (NW, SPW), 0) >= NW // B, ROWS, 0)
    hi32 = boff + jnp.maximum(e + 1, 1)
    lo32 = boff + jnp.maximum(lo, 1)
    idxa_ref[...] = jnp.concatenate([hi32[:, :HALF], lo32[:, :HALF]], axis=1)
    idxb_ref[...] = jnp.concatenate([hi32[:, HALF:], lo32[:, HALF:]], axis=1)

    eb = enb_ref[...] - 1
    wb = eb - stb_ref[...]
    lcntb = jnp.where(wb >= 0, wb + 1, wmax)
    lob = eb - lcntb + 1
    c0b = jnp.maximum(0, jnp.minimum(eb, 0) - lob + 1)
    invb = 1.0 / lcntb.astype(jnp.float32)
    ag_ref[...] = jnp.concatenate([invb, c0b.astype(jnp.float32) * invb],
                                  axis=1)


def _sc_body(p_hbm, idxa_hbm, idxb_hbm, ag_hbm, out_hbm,
             idxa_v, idxb_v, rows_a, rows_b, out_a, out_b, ag_v, seq0_v,
             sem_a, sem_b, sem_oa, sem_ob):
    wid = lax.axis_index("s") * 2 + lax.axis_index("c")
    base = wid * SPW
    pltpu.sync_copy(idxa_hbm.at[wid], idxa_v)
    pltpu.sync_copy(idxb_hbm.at[wid], idxb_v)
    cpa = pltpu.async_copy(p_hbm.at[idxa_v], rows_a, sem_a)
    cpb = pltpu.async_copy(p_hbm.at[idxb_v], rows_b, sem_b)
    pltpu.sync_copy(ag_hbm.at[wid], ag_v)
    row0 = jnp.where(wid < NW // B, 1, ROWS + 1)
    pltpu.sync_copy(p_hbm.at[pl.ds(row0, 1)], seq0_v)

    def make_chunk(rows_v, out_v, joff):
        def span_body(j, carry):
            av = ag_v[joff + j, :]
            gv = ag_v[SPW + joff + j, :]
            for kk in range(D // 16):
                sl = pl.ds(kk * 16, 16)
                h = rows_v[j, sl]
                l = rows_v[HALF + j, sl]
                out_v[j, sl] = av * (h - l) + gv * seq0_v[0, sl]
            return carry
        return span_body

    cpa.wait()
    lax.fori_loop(0, HALF, make_chunk(rows_a, out_a, 0), 0)
    cpoa = pltpu.async_copy(out_a, out_hbm.at[pl.ds(base, HALF)], sem_oa)
    cpb.wait()
    lax.fori_loop(0, HALF, make_chunk(rows_b, out_b, HALF), 0)
    cpob = pltpu.async_copy(out_b, out_hbm.at[pl.ds(base + HALF, HALF)],
                            sem_ob)
    cpoa.wait()
    cpob.wait()


@jax.jit
def kernel(sequence_tensor, span_indices):
    seq = sequence_tensor[:, :ROWS, :].reshape(B * ROWS, D)
    sp = span_indices.astype(jnp.int32)
    starts = sp[..., 0].reshape(NW, SPW)
    ends = sp[..., 1].reshape(NW, SPW)
    starts_bc = jnp.broadcast_to(sp[..., 0].reshape(NW, SPW, 1),
                                 (NW, SPW, 16))
    ends_bc = jnp.broadcast_to(sp[..., 1].reshape(NW, SPW, 1), (NW, SPW, 16))

    p_tab, idxa, idxb, ag = pl.pallas_call(
        _prep_body,
        out_shape=(
            jax.ShapeDtypeStruct((B * ROWS, D), jnp.float32),
            jax.ShapeDtypeStruct((NW, SPW), jnp.int32),
            jax.ShapeDtypeStruct((NW, SPW), jnp.int32),
            jax.ShapeDtypeStruct((NW, 2 * SPW, 16), jnp.float32),
        ),
    )(seq, starts, ends, starts_bc, ends_bc)

    sc_fn = functools.partial(
        pl.kernel,
        out_type=jax.ShapeDtypeStruct((NSP, D), jnp.float32),
        mesh=plsc.VectorSubcoreMesh(core_axis_name="c", subcore_axis_name="s"),
        scratch_types=[
            pltpu.VMEM((SPW,), jnp.int32),
            pltpu.VMEM((SPW,), jnp.int32),
            pltpu.VMEM((SPW, D), jnp.float32),
            pltpu.VMEM((SPW, D), jnp.float32),
            pltpu.VMEM((HALF, D), jnp.float32),
            pltpu.VMEM((HALF, D), jnp.float32),
            pltpu.VMEM((2 * SPW, 16), jnp.float32),
            pltpu.VMEM((1, D), jnp.float32),
            pltpu.SemaphoreType.DMA,
            pltpu.SemaphoreType.DMA,
            pltpu.SemaphoreType.DMA,
            pltpu.SemaphoreType.DMA,
        ],
    )(_sc_body)

    out = sc_fn(p_tab, idxa, idxb, ag)
    return out.reshape(B, NSPAN, D)


# speedup vs baseline: 1.4158x; 1.4158x over previous
import functools

import jax
import jax.numpy as jnp
from jax import lax
from jax.experimental import pallas as pl
from jax.experimental.pallas import tpu as pltpu
from jax.experimental.pallas import tpu_sc as plsc

B = 2
NSPAN = 512
D = 1024
ROWS = 64
TROWS = 2 * ROWS
NSP = B * NSPAN
NW = 32
SPW = NSP // NW
HALF = SPW // 2


def _prep_body(seq_ref, st_ref, en_ref, stb_ref, enb_ref,
               t_ref, idxa_ref, idxb_ref, a_ref):
    i = lax.broadcasted_iota(jnp.int32, (B * TROWS, B * ROWS), 0) % TROWS
    p = lax.broadcasted_iota(jnp.int32, (B * TROWS, B * ROWS), 1) % ROWS
    sameb = (lax.broadcasted_iota(jnp.int32, (B * TROWS, B * ROWS), 0)
             // TROWS) == (
        lax.broadcasted_iota(jnp.int32, (B * TROWS, B * ROWS), 1) // ROWS)
    m = jnp.where(p == 0, jnp.minimum(i, 65), (p < i - 64).astype(jnp.int32))
    mat = jnp.where(sameb, m, 0).astype(jnp.float32)
    t_ref[...] = lax.dot_general(
        mat, seq_ref[...], (((1,), (0,)), ((), ())),
        preferred_element_type=jnp.float32)

    e = en_ref[...] - 1
    w = e - st_ref[...]
    wmax = jnp.max(w) + 1
    lcnt = jnp.where(w >= 0, w + 1, wmax)
    boff = jnp.where(
        lax.broadcasted_iota(jnp.int32, (NW, SPW), 0) >= NW // B, TROWS, 0)
    hi32 = boff + e + 65
    lo32 = boff + e - lcnt + 65
    idxa_ref[...] = jnp.concatenate([hi32[:, :HALF], lo32[:, :HALF]], axis=1)
    idxb_ref[...] = jnp.concatenate([hi32[:, HALF:], lo32[:, HALF:]], axis=1)

    eb = enb_ref[...] - 1
    wb = eb - stb_ref[...]
    lcntb = jnp.where(wb >= 0, wb + 1, wmax)
    a_ref[...] = 1.0 / lcntb.astype(jnp.float32)


def _sc_body(t_hbm, idxa_hbm, idxb_hbm, a_hbm, out_hbm,
             idxa_v, idxb_v, rows_a, rows_b, out_a, out_b, a_v,
             sem_a, sem_b, sem_oa, sem_ob):
    wid = lax.axis_index("s") * 2 + lax.axis_index("c")
    base = wid * SPW
    pltpu.sync_copy(idxa_hbm.at[wid], idxa_v)
    pltpu.sync_copy(idxb_hbm.at[wid], idxb_v)
    cpa = pltpu.async_copy(t_hbm.at[idxa_v], rows_a, sem_a)
    cpb = pltpu.async_copy(t_hbm.at[idxb_v], rows_b, sem_b)
    pltpu.sync_copy(a_hbm.at[wid], a_v)

    def make_chunk(rows_v, out_v, joff):
        def span_body(j, carry):
            av = a_v[joff + j, :]
            for kk in range(D // 16):
                sl = pl.ds(kk * 16, 16)
                out_v[j, sl] = av * (rows_v[j, sl] - rows_v[HALF + j, sl])
            return carry
        return span_body

    cpa.wait()
    lax.fori_loop(0, HALF, make_chunk(rows_a, out_a, 0), 0)
    cpoa = pltpu.async_copy(out_a, out_hbm.at[pl.ds(base, HALF)], sem_oa)
    cpb.wait()
    lax.fori_loop(0, HALF, make_chunk(rows_b, out_b, HALF), 0)
    cpob = pltpu.async_copy(out_b, out_hbm.at[pl.ds(base + HALF, HALF)],
                            sem_ob)
    cpoa.wait()
    cpob.wait()


@jax.jit
def kernel(sequence_tensor, span_indices):
    seq = sequence_tensor[:, :ROWS, :].reshape(B * ROWS, D)
    sp = span_indices.astype(jnp.int32)
    starts = sp[..., 0].reshape(NW, SPW)
    ends = sp[..., 1].reshape(NW, SPW)
    starts_bc = jnp.broadcast_to(sp[..., 0].reshape(NW, SPW, 1),
                                 (NW, SPW, 16))
    ends_bc = jnp.broadcast_to(sp[..., 1].reshape(NW, SPW, 1), (NW, SPW, 16))

    t_tab, idxa, idxb, a_bc = pl.pallas_call(
        _prep_body,
        out_shape=(
            jax.ShapeDtypeStruct((B * TROWS, D), jnp.float32),
            jax.ShapeDtypeStruct((NW, SPW), jnp.int32),
            jax.ShapeDtypeStruct((NW, SPW), jnp.int32),
            jax.ShapeDtypeStruct((NW, SPW, 16), jnp.float32),
        ),
    )(seq, starts, ends, starts_bc, ends_bc)

    sc_fn = functools.partial(
        pl.kernel,
        out_type=jax.ShapeDtypeStruct((NSP, D), jnp.float32),
        mesh=plsc.VectorSubcoreMesh(core_axis_name="c", subcore_axis_name="s"),
        scratch_types=[
            pltpu.VMEM((SPW,), jnp.int32),
            pltpu.VMEM((SPW,), jnp.int32),
            pltpu.VMEM((SPW, D), jnp.float32),
            pltpu.VMEM((SPW, D), jnp.float32),
            pltpu.VMEM((HALF, D), jnp.float32),
            pltpu.VMEM((HALF, D), jnp.float32),
            pltpu.VMEM((SPW, 16), jnp.float32),
            pltpu.SemaphoreType.DMA,
            pltpu.SemaphoreType.DMA,
            pltpu.SemaphoreType.DMA,
            pltpu.SemaphoreType.DMA,
        ],
    )(_sc_body)

    out = sc_fn(t_tab, idxa, idxb, a_bc)
    return out.reshape(B, NSPAN, D)


# speedup vs baseline: 1.4158x; 1.0000x over previous
import functools

import jax
import jax.numpy as jnp
from jax import lax
from jax.experimental import pallas as pl
from jax.experimental.pallas import tpu as pltpu
from jax.experimental.pallas import tpu_sc as plsc

B = 2
NSPAN = 512
D = 1024
ROWS = 64
TROWS = 2 * ROWS
NSP = B * NSPAN
NW = 32
SPW = NSP // NW
NCH = 4
CH = SPW // NCH


def _prep_body(seq_ref, st_ref, en_ref, t_ref, idx_ref, a_ref):
    i = lax.broadcasted_iota(jnp.int32, (B * TROWS, B * ROWS), 0) % TROWS
    p = lax.broadcasted_iota(jnp.int32, (B * TROWS, B * ROWS), 1) % ROWS
    sameb = (lax.broadcasted_iota(jnp.int32, (B * TROWS, B * ROWS), 0)
             // TROWS) == (
        lax.broadcasted_iota(jnp.int32, (B * TROWS, B * ROWS), 1) // ROWS)
    m = jnp.where(p == 0, jnp.minimum(i, 65), (p < i - 64).astype(jnp.int32))
    mat = jnp.where(sameb, m, 0).astype(jnp.float32)
    t_ref[...] = lax.dot_general(
        mat, seq_ref[...].reshape(B * ROWS, D), (((1,), (0,)), ((), ())),
        preferred_element_type=jnp.float32)

    e = en_ref[...] - 1
    w = e - st_ref[...]
    wmax = jnp.max(w) + 1
    lcnt = jnp.where(w >= 0, w + 1, wmax)
    boff = jnp.where(
        lax.broadcasted_iota(jnp.int32, (NW, SPW), 0) >= NW // B, TROWS, 0)
    hi32 = boff + e + 65
    lo32 = boff + e - lcnt + 65
    idx_ref[...] = jnp.concatenate(
        [jnp.concatenate([hi32[:, c * CH:(c + 1) * CH],
                          lo32[:, c * CH:(c + 1) * CH]], axis=1)
         for c in range(NCH)], axis=1)
    a_ref[...] = 1.0 / lcnt.astype(jnp.float32)


def _sc_body(t_hbm, idx_hbm, a_hbm, out_hbm,
             idx_v, r0, r1, r2, r3, o0, o1, o2, o3, a_v,
             sg0, sg1, sg2, sg3, so0, so1, so2, so3):
    rows = [r0, r1, r2, r3]
    outs = [o0, o1, o2, o3]
    sgs = [sg0, sg1, sg2, sg3]
    sos = [so0, so1, so2, so3]
    wid = lax.axis_index("s") * 2 + lax.axis_index("c")
    base = wid * SPW
    pltpu.sync_copy(idx_hbm.at[wid], idx_v)
    gathers = []
    for c in range(NCH):
        jvec = idx_v[pl.ds(c * 2 * CH, 16)]
        gathers.append(pltpu.async_copy(t_hbm.at[jvec], rows[c], sgs[c]))
    pltpu.sync_copy(a_hbm.at[wid], a_v)
    a16 = [a_v[pl.ds(0, 16)], a_v[pl.ds(16, 16)]]
    zero16 = jnp.zeros((16,), jnp.int32)

    stores = []
    for c in range(NCH):
        gathers[c].wait()
        avec = a16[c // 2]
        rc, oc = rows[c], outs[c]
        aoff = (c % 2) * CH

        def span_body(j, carry, avec=avec, rc=rc, oc=oc, aoff=aoff):
            av = avec.at[zero16 + (aoff + j)].get(mode="promise_in_bounds")
            for kk in range(D // 16):
                sl = pl.ds(kk * 16, 16)
                oc[j, sl] = av * (rc[j, sl] - rc[CH + j, sl])
            return carry

        lax.fori_loop(0, CH, span_body, 0)
        stores.append(pltpu.async_copy(
            outs[c], out_hbm.at[pl.ds(base + c * CH, CH)], sos[c]))
    for cp in stores:
        cp.wait()


@jax.jit
def kernel(sequence_tensor, span_indices):
    sp = span_indices.astype(jnp.int32)
    starts = sp[..., 0].reshape(NW, SPW)
    ends = sp[..., 1].reshape(NW, SPW)

    t_tab, idx_all, a_all = pl.pallas_call(
        _prep_body,
        grid=(1,),
        in_specs=[
            pl.BlockSpec((B, ROWS, D), lambda i: (0, 0, 0)),
            pl.BlockSpec((NW, SPW), lambda i: (0, 0)),
            pl.BlockSpec((NW, SPW), lambda i: (0, 0)),
        ],
        out_specs=(
            pl.BlockSpec((B * TROWS, D), lambda i: (0, 0)),
            pl.BlockSpec((NW, 2 * SPW), lambda i: (0, 0)),
            pl.BlockSpec((NW, SPW), lambda i: (0, 0)),
        ),
        out_shape=(
            jax.ShapeDtypeStruct((B * TROWS, D), jnp.float32),
            jax.ShapeDtypeStruct((NW, 2 * SPW), jnp.int32),
            jax.ShapeDtypeStruct((NW, SPW), jnp.float32),
        ),
    )(sequence_tensor, starts, ends)

    sc_fn = functools.partial(
        pl.kernel,
        out_type=jax.ShapeDtypeStruct((NSP, D), jnp.float32),
        mesh=plsc.VectorSubcoreMesh(core_axis_name="c", subcore_axis_name="s"),
        scratch_types=(
            [pltpu.VMEM((2 * SPW,), jnp.int32)]
            + [pltpu.VMEM((2 * CH, D), jnp.float32) for _ in range(NCH)]
            + [pltpu.VMEM((CH, D), jnp.float32) for _ in range(NCH)]
            + [pltpu.VMEM((SPW,), jnp.float32)]
            + [pltpu.SemaphoreType.DMA for _ in range(2 * NCH)]
        ),
    )(_sc_body)

    out = sc_fn(t_tab, idx_all, a_all)
    return out.reshape(B, NSPAN, D)


# speedup vs baseline: 1.9730x; 1.3935x over previous
import functools

import jax
import jax.numpy as jnp
from jax import lax
from jax.experimental import pallas as pl
from jax.experimental.pallas import tpu as pltpu
from jax.experimental.pallas import tpu_sc as plsc

B = 2
NSPAN = 512
D = 1024
ROWS = 64
TROWS = 2 * ROWS
NSP = B * NSPAN
NW = 32
SPW = NSP // NW
NCH = 4
CH = SPW // NCH


def _prep_body(seq_ref, st_ref, en_ref, t_ref, idx_ref, a_ref):
    i = lax.broadcasted_iota(jnp.int32, (B * TROWS, B * ROWS), 0) % TROWS
    p = lax.broadcasted_iota(jnp.int32, (B * TROWS, B * ROWS), 1) % ROWS
    sameb = (lax.broadcasted_iota(jnp.int32, (B * TROWS, B * ROWS), 0)
             // TROWS) == (
        lax.broadcasted_iota(jnp.int32, (B * TROWS, B * ROWS), 1) // ROWS)
    m = jnp.where(p == 0, jnp.minimum(i, 65), (p < i - 64).astype(jnp.int32))
    mat = jnp.where(sameb, m, 0).astype(jnp.float32)
    t_ref[...] = lax.dot_general(
        mat, seq_ref[...].reshape(B * ROWS, D), (((1,), (0,)), ((), ())),
        preferred_element_type=jnp.float32)

    e = en_ref[...] - 1
    w = e - st_ref[...]
    wmax = jnp.max(w) + 1
    lcnt = jnp.where(w >= 0, w + 1, wmax)
    boff = jnp.where(
        lax.broadcasted_iota(jnp.int32, (NW, SPW), 0) >= NW // B, TROWS, 0)
    hi32 = boff + e + 65
    lo32 = boff + e - lcnt + 65
    idx_ref[...] = jnp.concatenate(
        [jnp.concatenate([hi32[:, c * CH:(c + 1) * CH],
                          lo32[:, c * CH:(c + 1) * CH]], axis=1)
         for c in range(NCH)], axis=1)
    a_ref[...] = 1.0 / lcnt.astype(jnp.float32)


def _sc_body(t_hbm, idx_hbm, a_hbm, out_hbm,
             idx_v, r0, r1, r2, r3, o0, o1, o2, o3, a_v,
             sg0, sg1, sg2, sg3, so0, so1, so2, so3):
    rows = [r0, r1, r2, r3]
    outs = [o0, o1, o2, o3]
    sgs = [sg0, sg1, sg2, sg3]
    sos = [so0, so1, so2, so3]
    wid = lax.axis_index("s") * 2 + lax.axis_index("c")
    base = wid * SPW
    pltpu.sync_copy(idx_hbm.at[wid], idx_v)
    gathers = []
    for c in range(NCH):
        jvec = idx_v[pl.ds(c * 2 * CH, 16)]
        gathers.append(pltpu.async_copy(t_hbm.at[jvec], rows[c], sgs[c]))
    pltpu.sync_copy(a_hbm.at[wid], a_v)
    a16 = [a_v[pl.ds(0, 16)], a_v[pl.ds(16, 16)]]
    zero16 = jnp.zeros((16,), jnp.int32)

    stores = []
    for c in range(NCH):
        gathers[c].wait()
        avec = a16[c // 2]
        rc, oc = rows[c], outs[c]
        aoff = (c % 2) * CH

        def span_body(j, carry, avec=avec, rc=rc, oc=oc, aoff=aoff):
            av = avec.at[zero16 + (aoff + j)].get(mode="promise_in_bounds")

            @plsc.parallel_loop(0, D // 16, unroll=8)
            def _(kk):
                sl = pl.ds(kk * 16, 16)
                oc[j, sl] = av * (rc[j, sl] - rc[CH + j, sl])

            return carry

        lax.fori_loop(0, CH, span_body, 0)
        stores.append(pltpu.async_copy(
            outs[c], out_hbm.at[pl.ds(base + c * CH, CH)], sos[c]))
    for cp in stores:
        cp.wait()


@jax.jit
def kernel(sequence_tensor, span_indices):
    sp = span_indices.astype(jnp.int32)
    starts = sp[..., 0].reshape(NW, SPW)
    ends = sp[..., 1].reshape(NW, SPW)

    t_tab, idx_all, a_all = pl.pallas_call(
        _prep_body,
        grid=(1,),
        in_specs=[
            pl.BlockSpec((B, ROWS, D), lambda i: (0, 0, 0)),
            pl.BlockSpec((NW, SPW), lambda i: (0, 0)),
            pl.BlockSpec((NW, SPW), lambda i: (0, 0)),
        ],
        out_specs=(
            pl.BlockSpec((B * TROWS, D), lambda i: (0, 0)),
            pl.BlockSpec((NW, 2 * SPW), lambda i: (0, 0)),
            pl.BlockSpec((NW, SPW), lambda i: (0, 0)),
        ),
        out_shape=(
            jax.ShapeDtypeStruct((B * TROWS, D), jnp.float32),
            jax.ShapeDtypeStruct((NW, 2 * SPW), jnp.int32),
            jax.ShapeDtypeStruct((NW, SPW), jnp.float32),
        ),
    )(sequence_tensor, starts, ends)

    sc_fn = functools.partial(
        pl.kernel,
        out_type=jax.ShapeDtypeStruct((NSP, D), jnp.float32),
        mesh=plsc.VectorSubcoreMesh(core_axis_name="c", subcore_axis_name="s"),
        scratch_types=(
            [pltpu.VMEM((2 * SPW,), jnp.int32)]
            + [pltpu.VMEM((2 * CH, D), jnp.float32) for _ in range(NCH)]
            + [pltpu.VMEM((CH, D), jnp.float32) for _ in range(NCH)]
            + [pltpu.VMEM((SPW,), jnp.float32)]
            + [pltpu.SemaphoreType.DMA for _ in range(2 * NCH)]
        ),
    )(_sc_body)

    out = sc_fn(t_tab, idx_all, a_all)
    return out.reshape(B, NSPAN, D)
